# Initial kernel scaffold; baseline (speedup 1.0000x reference)
import jax
import jax.numpy as jnp
from jax.experimental import pallas as pl


def kernel(x, edge_index, edge_attr, batch, atom_emb, bond_emb, W, b, gamma, beta, W1, b1, W2, b2):
    raise NotImplementedError("write your pallas kernel here")


# speedup vs baseline: 7.3974x; 7.3974x over previous
import functools

import jax
import jax.numpy as jnp
from jax import lax
from jax.experimental import pallas as pl
from jax.experimental.pallas import tpu as pltpu
from jax.experimental.pallas import tpu_sc as plsc

NN = 10000
NE = 320000
HD = 128
NL = 4
NG = 64
OUTD = 128

NWORK = 32
K = 128
CHUNKS = 79
EW = K * CHUNKS
E_PAD = NWORK * EW
N_ACC = 10240
RPT = N_ACC // 16
JUNK = NN + 8

_HIGH = lax.Precision.HIGHEST


def _sc_segment_sum():
  mesh = plsc.VectorSubcoreMesh(core_axis_name="c", subcore_axis_name="s")
  scratch = [
      pltpu.VMEM((K,), jnp.int32),
      pltpu.VMEM((K,), jnp.int32),
      pltpu.VMEM((K, HD), jnp.float32),
      pltpu.VMEM((K, HD), jnp.float32),
      pltpu.VMEM_SHARED((N_ACC, HD), jnp.float32),
      pltpu.SemaphoreType.DMA,
  ]

  def entry(table_hbm, src_hbm, dst_hbm, zero_hbm, out_hbm, *refs):
    src_v, dst_v, rows_v, zbuf, acc, sem = refs
    core = lax.axis_index("c")
    sub = lax.axis_index("s")
    wid = sub * 2 + core
    rb = sub * RPT
    pltpu.sync_copy(zero_hbm, zbuf)
    for j in range(RPT // K):
      pltpu.sync_copy(zbuf, acc.at[pl.ds(rb + j * K, K)])
    plsc.subcore_barrier()
    base0 = wid * EW

    def step(g, carry):
      bs = pl.multiple_of(base0 + g * K, 8)
      pltpu.sync_copy(dst_hbm.at[pl.ds(bs, K)], dst_v)
      pltpu.sync_copy(src_hbm.at[pl.ds(bs, K)], src_v)
      pltpu.async_copy(table_hbm.at[src_v], rows_v, sem).wait()
      pltpu.sync_copy(rows_v, acc.at[dst_v], add=True)
      return carry

    lax.fori_loop(0, CHUNKS, step, 0)
    plsc.subcore_barrier()
    for j in range(RPT // K):
      o = pl.multiple_of(rb + j * K, 8)
      pltpu.sync_copy(acc.at[pl.ds(o, K)], rows_v)
      pltpu.sync_copy(rows_v, out_hbm.at[core, pl.ds(o, K)])

  return functools.partial(
      pl.kernel, mesh=mesh,
      out_type=jax.ShapeDtypeStruct((2, N_ACC, HD), jnp.float32),
      scratch_types=scratch)(entry)


_sc_gather_add = _sc_segment_sum()


def _tc_encode(xf, da, t0):
  def body(x_ref, d_ref, t_ref, o_ref):
    o_ref[...] = jnp.dot(x_ref[...], d_ref[...], precision=_HIGH,
                         preferred_element_type=jnp.float32) + t_ref[...]
  return pl.pallas_call(
      body, out_shape=jax.ShapeDtypeStruct((NN, HD), jnp.float32))(xf, da, t0)


def _layer_core(p_ref, a_ref, h_ref, dl_ref, b0_ref, w_ref, bi_ref, g_ref,
                be_ref):
  aa = a_ref[0, :NN, :] + a_ref[1, :NN, :]
  bond = jnp.dot(aa, dl_ref[...], precision=_HIGH,
                 preferred_element_type=jnp.float32) + b0_ref[...]
  agg = p_ref[0, :NN, :] + p_ref[1, :NN, :] + h_ref[...] + bond
  z = jnp.dot(agg, w_ref[...],
              preferred_element_type=jnp.float32) + bi_ref[...]
  mu = jnp.mean(z, axis=0, keepdims=True)
  var = jnp.mean(jnp.square(z - mu), axis=0, keepdims=True)
  zn = (z - mu) * lax.rsqrt(var + 1e-5) * g_ref[...] + be_ref[...]
  return jnp.maximum(zn, 0.0)


def _tc_layer(p, dA, h, dl, b0, w, bias, gam, bet):
  def body(p_ref, a_ref, h_ref, dl_ref, b0_ref, w_ref, bi_ref, g_ref, be_ref,
           o_ref):
    o_ref[...] = _layer_core(p_ref, a_ref, h_ref, dl_ref, b0_ref, w_ref,
                             bi_ref, g_ref, be_ref)
  return pl.pallas_call(
      body, out_shape=jax.ShapeDtypeStruct((NN, HD), jnp.float32))(
          p, dA, h, dl, b0, w, bias, gam, bet)


def _tc_final(p, dA, h, dl, b0, w, bias, gam, bet, batch_row, w1, b1r, w2,
              b2r):
  def body(p_ref, a_ref, h_ref, dl_ref, b0_ref, w_ref, bi_ref, g_ref, be_ref,
           bt_ref, w1_ref, b1_ref, w2_ref, b2_ref, h_out, pred_out):
    h4 = _layer_core(p_ref, a_ref, h_ref, dl_ref, b0_ref, w_ref, bi_ref,
                     g_ref, be_ref)
    h_out[...] = h4
    gid = lax.broadcasted_iota(jnp.int32, (NG, 1), 0)
    m = (bt_ref[...] == gid).astype(jnp.float32)
    gsum = jnp.dot(m, h4, precision=_HIGH,
                   preferred_element_type=jnp.float32)
    cnt = jnp.sum(m, axis=1, keepdims=True)
    gmean = gsum / jnp.maximum(cnt, 1.0)
    act = jnp.maximum(
        jnp.dot(gmean, w1_ref[...],
                preferred_element_type=jnp.float32) + b1_ref[...], 0.0)
    pred_out[...] = jnp.dot(act, w2_ref[...],
                            preferred_element_type=jnp.float32) + b2_ref[...]

  return pl.pallas_call(
      body, out_shape=(jax.ShapeDtypeStruct((NN, HD), jnp.float32),
                       jax.ShapeDtypeStruct((NG, OUTD), jnp.float32)))(
          p, dA, h, dl, b0, w, bias, gam, bet, batch_row, w1, b1r, w2, b2r)


def kernel(x, edge_index, edge_attr, batch, atom_emb, bond_emb, W, b, gamma,
           beta, W1, b1, W2, b2):
  f32 = jnp.float32
  pad = E_PAD - NE
  src_p = jnp.concatenate([edge_index[0].astype(jnp.int32),
                           jnp.zeros((pad,), jnp.int32)])
  dst_p = jnp.concatenate([edge_index[1].astype(jnp.int32),
                           jnp.full((pad,), JUNK, jnp.int32)])
  code = jnp.sum(edge_attr.astype(jnp.int32) * (2 ** jnp.arange(5))[None, :],
                 axis=1)
  code_p = jnp.concatenate([code, jnp.zeros((pad,), jnp.int32)])
  cbits = ((jnp.arange(32)[:, None] >> jnp.arange(5)[None, :]) & 1).astype(f32)
  tind = jnp.concatenate(
      [jnp.ones((32, 1), f32), cbits, jnp.zeros((32, HD - 6), f32)], axis=1)
  zero_h = jnp.zeros((K, HD), f32)

  xf = jnp.concatenate([x.astype(f32), jnp.zeros((NN, 16 - 9), f32)], axis=1)
  da = jnp.concatenate([atom_emb[:, 1, :] - atom_emb[:, 0, :],
                        jnp.zeros((16 - 9, HD), f32)], axis=0)
  t0 = jnp.sum(atom_emb[:, 0, :], axis=0, keepdims=True)

  b0 = jnp.sum(bond_emb[:, :, 0, :], axis=1)
  db = bond_emb[:, :, 1, :] - bond_emb[:, :, 0, :]
  dl_all = jnp.concatenate(
      [b0[:, None, :], db, jnp.zeros((NL, HD - 6, HD), f32)], axis=1)

  batch_row = batch.astype(jnp.int32).reshape(1, NN)
  brow = b.reshape(NL, 1, HD)
  grow = gamma.reshape(NL, 1, HD)
  berow = beta.reshape(NL, 1, HD)
  b1r = b1.reshape(1, -1)
  b2r = b2.reshape(1, -1)

  dA = _sc_gather_add(tind, code_p, dst_p, zero_h)
  h = _tc_encode(xf, da, t0)
  pred = None
  for l in range(NL):
    p = _sc_gather_add(h, src_p, dst_p, zero_h)
    if l < NL - 1:
      h = _tc_layer(p, dA, h, dl_all[l], b0[l:l + 1], W[l], brow[l], grow[l],
                    berow[l])
    else:
      h, pred = _tc_final(p, dA, h, dl_all[l], b0[l:l + 1], W[l], brow[l],
                          grow[l], berow[l], batch_row, W1, b1r, W2, b2r)
  return (pred, h)
